# Initial kernel scaffold; baseline (speedup 1.0000x reference)
import jax
import jax.numpy as jnp
from jax.experimental import pallas as pl


def kernel(x, edge_index, ln_w, ln_b, W_l, W_r, att, bias):
    raise NotImplementedError("write your pallas kernel here")


# speedup vs baseline: 1.6480x; 1.6480x over previous
import functools

import jax
import jax.numpy as jnp
from jax import lax
from jax.experimental import pallas as pl

N = 10000
E = 160000
D = 256
ROW_BLK = 400


def _ln_proj_body(x_ref, lnw_ref, lnb_ref, wl_ref, wr_ref, xl0_ref, xl1_ref, xr_ref):
    xb = x_ref[...]
    mu = jnp.mean(xb, axis=-1, keepdims=True)
    var = jnp.mean((xb - mu) ** 2, axis=-1, keepdims=True)
    xn = (xb - mu) / jnp.sqrt(var + 1e-5) * lnw_ref[...] + lnb_ref[...]
    xl = jnp.dot(xn, wl_ref[...], preferred_element_type=jnp.float32)
    xr = jnp.dot(xn, wr_ref[...], preferred_element_type=jnp.float32)
    xl0_ref[...] = xl[:, :128]
    xl1_ref[...] = xl[:, 128:]
    xr_ref[...] = xr


def _ln_proj(x, ln_w, ln_b, W_l, W_r):
    grid = (N // ROW_BLK,)
    return pl.pallas_call(
        _ln_proj_body,
        grid=grid,
        in_specs=[
            pl.BlockSpec((ROW_BLK, D), lambda i: (i, 0)),
            pl.BlockSpec((D,), lambda i: (0,)),
            pl.BlockSpec((D,), lambda i: (0,)),
            pl.BlockSpec((D, D), lambda i: (0, 0)),
            pl.BlockSpec((D, D), lambda i: (0, 0)),
        ],
        out_specs=[
            pl.BlockSpec((ROW_BLK, 128), lambda i: (i, 0)),
            pl.BlockSpec((ROW_BLK, 128), lambda i: (i, 0)),
            pl.BlockSpec((ROW_BLK, D), lambda i: (i, 0)),
        ],
        out_shape=[
            jax.ShapeDtypeStruct((N, 128), jnp.float32),
            jax.ShapeDtypeStruct((N, 128), jnp.float32),
            jax.ShapeDtypeStruct((N, D), jnp.float32),
        ],
    )(x, ln_w, ln_b, W_l, W_r)


def _finish_body(n0_ref, n1_ref, d_ref, b_ref, out_ref):
    num = jnp.concatenate([n0_ref[...], n1_ref[...]], axis=1)
    den = d_ref[...][:, :1]
    out = num / (den + 1e-16) + b_ref[...]
    out_ref[...] = 0.5 * out * (1.0 + lax.erf(out / jnp.sqrt(2.0).astype(jnp.float32)))


def _finish(num0, num1, den, bias):
    grid = (N // ROW_BLK,)
    return pl.pallas_call(
        _finish_body,
        grid=grid,
        in_specs=[
            pl.BlockSpec((ROW_BLK, 128), lambda i: (i, 0)),
            pl.BlockSpec((ROW_BLK, 128), lambda i: (i, 0)),
            pl.BlockSpec((ROW_BLK, 8), lambda i: (i, 0)),
            pl.BlockSpec((D,), lambda i: (0,)),
        ],
        out_specs=pl.BlockSpec((ROW_BLK, D), lambda i: (i, 0)),
        out_shape=jax.ShapeDtypeStruct((N, D), jnp.float32),
    )(num0, num1, den, bias)


def kernel(x, edge_index, ln_w, ln_b, W_l, W_r, att, bias):
    xl0, xl1, xr = _ln_proj(x, ln_w, ln_b, W_l, W_r)
    xl = jnp.concatenate([xl0, xl1], axis=1)

    loops = jnp.arange(N, dtype=edge_index.dtype)
    src = jnp.concatenate([edge_index[0], loops])
    dst = jnp.concatenate([edge_index[1], loops])

    e = jax.nn.leaky_relu(xl[src] + xr[dst], 0.2)
    logit = (e * att[0]).sum(-1)
    M = logit.max()
    w = jnp.exp(logit - M)
    num = jax.ops.segment_sum(xl[src] * w[:, None], dst, num_segments=N)
    den = jax.ops.segment_sum(w, dst, num_segments=N)
    den8 = jnp.broadcast_to(den[:, None], (N, 8))

    return _finish(num[:, :128], num[:, 128:], den8, bias)


# speedup vs baseline: 1.9440x; 1.1796x over previous
import functools

import jax
import jax.numpy as jnp
from jax import lax
from jax.experimental import pallas as pl
from jax.experimental.pallas import tpu as pltpu
from jax.experimental.pallas import tpu_sc as plsc

N = 10000
E = 160000
ETOT = E + N
D = 256
HD = 128
ROW_BLK = 400

NC = 2
NS = 16
NW = NC * NS
EB = 64
EPAD = 180224
RPT_A = EPAD // (NW * EB)
RPT_B = EPAD // (NS * EB)
NPAD = 10240
NPT = NPAD // NS
NCHUNK = 32
GB = 8
NGRP = RPT_B // GB
DW = HD + 16
_SC_PARAMS = None

_MESH = plsc.VectorSubcoreMesh(
    core_axis_name="c", subcore_axis_name="s", num_cores=NC, num_subcores=NS)
_SC_PARAMS = pltpu.CompilerParams(use_tc_tiling_on_sc=False)


def _ln_proj_body(x_ref, lnw_ref, lnb_ref, wl_ref, wr_ref, xl0_ref, xl1_ref, xr_ref):
    xb = x_ref[...]
    mu = jnp.mean(xb, axis=-1, keepdims=True)
    var = jnp.mean((xb - mu) ** 2, axis=-1, keepdims=True)
    xn = (xb - mu) / jnp.sqrt(var + 1e-5) * lnw_ref[...] + lnb_ref[...]
    xl = jnp.dot(xn, wl_ref[...], preferred_element_type=jnp.float32)
    xr = jnp.dot(xn, wr_ref[...], preferred_element_type=jnp.float32)
    xl0_ref[...] = xl[:, :HD]
    xl1_ref[...] = xl[:, HD:]
    xr_ref[...] = xr


def _ln_proj(x, ln_w, ln_b, W_l, W_r):
    return pl.pallas_call(
        _ln_proj_body,
        grid=(N // ROW_BLK,),
        in_specs=[
            pl.BlockSpec((ROW_BLK, D), lambda i: (i, 0)),
            pl.BlockSpec((D,), lambda i: (0,)),
            pl.BlockSpec((D,), lambda i: (0,)),
            pl.BlockSpec((D, D), lambda i: (0, 0)),
            pl.BlockSpec((D, D), lambda i: (0, 0)),
        ],
        out_specs=[
            pl.BlockSpec((ROW_BLK, HD), lambda i: (i, 0)),
            pl.BlockSpec((ROW_BLK, HD), lambda i: (i, 0)),
            pl.BlockSpec((ROW_BLK, D), lambda i: (i, 0)),
        ],
        out_shape=[
            jax.ShapeDtypeStruct((N, HD), jnp.float32),
            jax.ShapeDtypeStruct((N, HD), jnp.float32),
            jax.ShapeDtypeStruct((N, D), jnp.float32),
        ],
    )(x, ln_w, ln_b, W_l, W_r)


def _lane_reduce(v, op):
    for sh in (8, 4, 2, 1):
        idx = jnp.arange(16, dtype=jnp.int32) ^ sh
        v = op(v, v.at[idx].get(mode="promise_in_bounds"))
    return v


def _lane_sum(v):
    return _lane_reduce(v, jnp.add)[0]


def _lane_max(v):
    return _lane_reduce(v, jnp.maximum)[0]

def _logits_body(xl0, xl1, xr, att_hbm, src2d, dst2d,
                 logits2d, maxes,
                 sidx_v, didx_v, l0_v, l1_v, r_v, att_v, log_v, max_v, sem):
    c = lax.axis_index("c")
    s = lax.axis_index("s")
    wid = s * NC + c
    rowbase = wid * RPT_A

    pltpu.sync_copy(src2d.at[pl.ds(rowbase, RPT_A)], sidx_v)
    pltpu.sync_copy(dst2d.at[pl.ds(rowbase, RPT_A)], didx_v)
    pltpu.sync_copy(att_hbm, att_v)
    att_regs = [att_v[pl.ds(16 * k, 16)] for k in range(16)]

    lanes = lax.iota(jnp.int32, 16)

    def batch_step(j, run_max):
        pltpu.async_copy(xl0.at[sidx_v.at[j]], l0_v, sem).wait()
        pltpu.async_copy(xl1.at[sidx_v.at[j]], l1_v, sem).wait()
        pltpu.async_copy(xr.at[didx_v.at[j]], r_v, sem).wait()

        for g in range(EB // 16):
            def edge_step(e2, lvec):
                e = g * 16 + e2
                acc = jnp.zeros((16,), jnp.float32)
                for k in range(16):
                    if k < 8:
                        lv = l0_v[e, pl.ds(16 * k, 16)]
                    else:
                        lv = l1_v[e, pl.ds(16 * (k - 8), 16)]
                    sv = lv + r_v[e, pl.ds(16 * k, 16)]
                    sv = jnp.maximum(sv, 0.2 * sv)
                    acc = acc + sv * att_regs[k]
                val = _lane_sum(acc)
                gid = (rowbase + j) * EB + e
                val = jnp.where(gid < ETOT, val, jnp.float32(-1e30))
                return jnp.where(lanes == e2, val, lvec)

            lvec = lax.fori_loop(0, 16, edge_step, jnp.zeros((16,), jnp.float32))
            log_v[j, pl.ds(16 * g, 16)] = lvec
            run_max = jnp.maximum(run_max, _lane_max(lvec))
        return run_max

    run_max = lax.fori_loop(0, RPT_A, batch_step, jnp.float32(-1e30))
    pltpu.sync_copy(log_v, logits2d.at[pl.ds(rowbase, RPT_A)])
    for k in range(8):
        max_v[pl.ds(16 * k, 16)] = jnp.full((16,), run_max)
    pltpu.sync_copy(max_v, maxes.at[pl.ds(wid * 128, 128)])


@functools.partial(
    pl.kernel,
    out_type=[
        jax.ShapeDtypeStruct((EPAD // EB, EB), jnp.float32),
        jax.ShapeDtypeStruct((NW * 128,), jnp.float32),
    ],
    mesh=_MESH,
    scratch_types=[
        pltpu.VMEM((RPT_A, EB), jnp.int32),
        pltpu.VMEM((RPT_A, EB), jnp.int32),
        pltpu.VMEM((EB, HD), jnp.float32),
        pltpu.VMEM((EB, HD), jnp.float32),
        pltpu.VMEM((EB, D), jnp.float32),
        pltpu.VMEM((D,), jnp.float32),
        pltpu.VMEM((RPT_A, EB), jnp.float32),
        pltpu.VMEM((128,), jnp.float32),
        pltpu.SemaphoreType.DMA,
    ],
)
def _logits_kernel(*refs):
    _logits_body(*refs)


def _agg_body(xlcat, logits2d, maxes, src2d, dst2d,
              out,
              accum,
              sidx_v, didx_v, log_v, l_v, ob_v, maxv, nv, sem):
    c = lax.axis_index("c")
    s = lax.axis_index("s")
    coff = c * N

    m = jnp.full((16,), jnp.float32(-1e30))
    for blk in range(NW * 128 // 1024):
        pltpu.sync_copy(maxes.at[pl.ds(blk * 1024, 1024)], maxv)
        for i in range(1024 // 128):
            m = jnp.maximum(m, maxv[pl.ds(128 * i, 16)])
    gmax = _lane_max(m)

    def zrow(r, _):
        for k in range(DW // 16):
            nv[r, pl.ds(16 * k, 16)] = jnp.zeros((16,), jnp.float32)
        return 0
    lax.fori_loop(0, NCHUNK, zrow, 0)
    for t in range(NPT // NCHUNK):
        pltpu.sync_copy(nv, accum.at[pl.ds(s * NPT + t * NCHUNK, NCHUNK)])

    onehot0 = jnp.where(lax.iota(jnp.int32, 16) == 0,
                        jnp.float32(1.0), jnp.float32(0.0))

    plsc.subcore_barrier()

    rowbase = s * RPT_B

    def group_step(g, _):
        gb = rowbase + g * GB
        pltpu.sync_copy(src2d.at[pl.ds(gb, GB)], sidx_v)
        pltpu.sync_copy(dst2d.at[pl.ds(gb, GB)], didx_v)
        pltpu.sync_copy(logits2d.at[pl.ds(gb, GB)], log_v)
        for r in range(GB):
            for k in range(EB // 16):
                sidx_v[r, pl.ds(16 * k, 16)] = sidx_v[r, pl.ds(16 * k, 16)] + coff

        def batch_step(j, _):
            pltpu.async_copy(xlcat.at[sidx_v.at[j]], l_v, sem).wait()

            wvs = [jnp.exp(log_v[j, pl.ds(16 * q, 16)] - gmax)
                   for q in range(EB // 16)]
            for e in range(EB):
                a = wvs[e >> 4][e & 15]
                for k in range(HD // 16):
                    ob_v[e, pl.ds(16 * k, 16)] = l_v[e, pl.ds(16 * k, 16)] * a
                ob_v[e, pl.ds(HD, 16)] = a * onehot0

            pltpu.sync_copy(ob_v, accum.at[didx_v.at[j]], add=True)
            return 0

        lax.fori_loop(0, GB, batch_step, 0)
        return 0

    lax.fori_loop(0, NGRP, group_step, 0)

    plsc.subcore_barrier()

    for t in range(NPT // NCHUNK):
        base = s * NPT + t * NCHUNK
        pltpu.sync_copy(accum.at[pl.ds(base, NCHUNK)], nv)
        pltpu.sync_copy(nv, out.at[pl.ds(c * NPAD + base, NCHUNK)])


@functools.partial(
    pl.kernel,
    out_type=jax.ShapeDtypeStruct((2 * NPAD, DW), jnp.float32),
    mesh=_MESH,
    compiler_params=_SC_PARAMS,
    scratch_types=[
        pltpu.VMEM_SHARED((NPAD, DW), jnp.float32),
        pltpu.VMEM((GB, EB), jnp.int32),
        pltpu.VMEM((GB, EB), jnp.int32),
        pltpu.VMEM((GB, EB), jnp.float32),
        pltpu.VMEM((EB, HD), jnp.float32),
        pltpu.VMEM((EB, DW), jnp.float32),
        pltpu.VMEM((1024,), jnp.float32),
        pltpu.VMEM((NCHUNK, DW), jnp.float32),
        pltpu.SemaphoreType.DMA,
    ],
)
def _agg_kernel(*refs):
    _agg_body(*refs)


def _finish_body(n0_ref, n1_ref, b_ref, out_ref):
    n0 = n0_ref[...]
    n1 = n1_ref[...]
    den = n0[:, HD:HD + 1] + jnp.float32(1e-16)
    out = jnp.concatenate([n0[:, :HD], n1[:, :HD]], axis=1) / den + b_ref[...]
    out_ref[...] = 0.5 * out * (1.0 + lax.erf(out / jnp.sqrt(2.0).astype(jnp.float32)))


def _finish(num0, num1, bias):
    return pl.pallas_call(
        _finish_body,
        grid=(N // ROW_BLK,),
        in_specs=[
            pl.BlockSpec((ROW_BLK, DW), lambda i: (i, 0)),
            pl.BlockSpec((ROW_BLK, DW), lambda i: (i, 0)),
            pl.BlockSpec((D,), lambda i: (0,)),
        ],
        out_specs=pl.BlockSpec((ROW_BLK, D), lambda i: (i, 0)),
        out_shape=jax.ShapeDtypeStruct((N, D), jnp.float32),
    )(num0, num1, bias)


def kernel(x, edge_index, ln_w, ln_b, W_l, W_r, att, bias):
    xl0, xl1, xr = _ln_proj(x, ln_w, ln_b, W_l, W_r)

    loops = jnp.arange(N, dtype=edge_index.dtype)
    pad = jnp.zeros((EPAD - ETOT,), dtype=edge_index.dtype)
    src2d = jnp.concatenate([edge_index[0], loops, pad]).reshape(EPAD // EB, EB)
    dst2d = jnp.concatenate([edge_index[1], loops, pad]).reshape(EPAD // EB, EB)

    logits2d, maxes = _logits_kernel(xl0, xl1, xr, att.reshape(D), src2d, dst2d)
    xlcat = jnp.concatenate([xl0, xl1], axis=0)
    outc = _agg_kernel(xlcat, logits2d, maxes, src2d, dst2d)
    return _finish(outc[:N], outc[NPAD:NPAD + N], bias)


# speedup vs baseline: 2.7771x; 1.4286x over previous
import functools

import jax
import jax.numpy as jnp
from jax import lax
from jax.experimental import pallas as pl
from jax.experimental.pallas import tpu as pltpu
from jax.experimental.pallas import tpu_sc as plsc

N = 10000
E = 160000
ETOT = E + N
D = 256
HD = 128
ROW_BLK = 400

NC = 2
NS = 16
NW = NC * NS
EB = 64
EPAD = 180224
RPT_A = EPAD // (NW * EB)
RPT_B = EPAD // (NS * EB)
NPAD = 10240
NPT = NPAD // NS
NCHUNK = 32
GB = 8
NGRP = RPT_B // GB
DW = HD + 16
_SC_PARAMS = None

_MESH = plsc.VectorSubcoreMesh(
    core_axis_name="c", subcore_axis_name="s", num_cores=NC, num_subcores=NS)
_SC_PARAMS = pltpu.CompilerParams(use_tc_tiling_on_sc=False)


def _ln_proj_body(x_ref, lnw_ref, lnb_ref, wl_ref, wr_ref, xl_ref, xr_ref):
    xb = x_ref[...]
    mu = jnp.mean(xb, axis=-1, keepdims=True)
    var = jnp.mean((xb - mu) ** 2, axis=-1, keepdims=True)
    xn = (xb - mu) / jnp.sqrt(var + 1e-5) * lnw_ref[...] + lnb_ref[...]
    xl_ref[...] = jnp.dot(xn, wl_ref[...], preferred_element_type=jnp.float32)
    xr_ref[...] = jnp.dot(xn, wr_ref[...], preferred_element_type=jnp.float32)


def _ln_proj(x, ln_w, ln_b, W_l, W_r):
    return pl.pallas_call(
        _ln_proj_body,
        grid=(N // ROW_BLK,),
        in_specs=[
            pl.BlockSpec((ROW_BLK, D), lambda i: (i, 0)),
            pl.BlockSpec((D,), lambda i: (0,)),
            pl.BlockSpec((D,), lambda i: (0,)),
            pl.BlockSpec((D, D), lambda i: (0, 0)),
            pl.BlockSpec((D, D), lambda i: (0, 0)),
        ],
        out_specs=[
            pl.BlockSpec((ROW_BLK, D), lambda i: (i, 0)),
            pl.BlockSpec((ROW_BLK, D), lambda i: (i, 0)),
        ],
        out_shape=[
            jax.ShapeDtypeStruct((N, D), jnp.float32),
            jax.ShapeDtypeStruct((N, D), jnp.float32),
        ],
    )(x, ln_w, ln_b, W_l, W_r)


def _lane_reduce(v, op):
    for sh in (8, 4, 2, 1):
        idx = jnp.arange(16, dtype=jnp.int32) ^ sh
        v = op(v, v.at[idx].get(mode="promise_in_bounds"))
    return v


def _lane_sum(v):
    return _lane_reduce(v, jnp.add)[0]


def _lane_max(v):
    return _lane_reduce(v, jnp.maximum)[0]

def _logits_body(xl, xr, att_hbm, src2d, dst2d,
                 logits2d, maxes,
                 sidx_v, didx_v, l_a, r_a, l_b, r_b,
                 att_v, log_v, max_v, sema, semb):
    c = lax.axis_index("c")
    s = lax.axis_index("s")
    wid = s * NC + c
    rowbase = wid * RPT_A

    pltpu.sync_copy(src2d.at[pl.ds(rowbase, RPT_A)], sidx_v)
    pltpu.sync_copy(dst2d.at[pl.ds(rowbase, RPT_A)], didx_v)
    pltpu.sync_copy(att_hbm, att_v)
    att_regs = [att_v[pl.ds(16 * k, 16)] for k in range(16)]

    lanes = lax.iota(jnp.int32, 16)
    bufs = ((l_a, r_a, sema), (l_b, r_b, semb))

    def fire(j, b):
        lv, rv, sem = bufs[b]
        pltpu.async_copy(xl.at[sidx_v.at[j]], lv, sem)
        pltpu.async_copy(xr.at[didx_v.at[j]], rv, sem)

    def drain(b):
        lv, rv, sem = bufs[b]
        pltpu.make_async_copy(xl.at[sidx_v.at[0]], lv, sem).wait()
        pltpu.make_async_copy(xr.at[didx_v.at[0]], rv, sem).wait()

    def compute(j, b, run_max):
        lbuf, rbuf, _ = bufs[b]
        for g in range(EB // 16):
            def edge_step(e2, lvec):
                e = g * 16 + e2
                acc = jnp.zeros((16,), jnp.float32)
                for k in range(16):
                    sv = lbuf[e, pl.ds(16 * k, 16)] + rbuf[e, pl.ds(16 * k, 16)]
                    sv = jnp.maximum(sv, 0.2 * sv)
                    acc = acc + sv * att_regs[k]
                val = _lane_sum(acc)
                gid = (rowbase + j) * EB + e
                val = jnp.where(gid < ETOT, val, jnp.float32(-1e30))
                return jnp.where(lanes == e2, val, lvec)

            lvec = lax.fori_loop(0, 16, edge_step, jnp.zeros((16,), jnp.float32))
            log_v[j, pl.ds(16 * g, 16)] = lvec
            run_max = jnp.maximum(run_max, _lane_max(lvec))
        return run_max

    fire(jnp.int32(0), 0)

    def outer(i, run_max):
        j0 = 2 * i
        fire(j0 + 1, 1)
        drain(0)
        run_max = compute(j0, 0, run_max)
        fire(jnp.minimum(j0 + 2, RPT_A - 1), 0)
        drain(1)
        run_max = compute(j0 + 1, 1, run_max)
        return run_max

    run_max = lax.fori_loop(0, RPT_A // 2, outer, jnp.float32(-1e30))
    drain(0)

    pltpu.sync_copy(log_v, logits2d.at[pl.ds(rowbase, RPT_A)])
    for k in range(8):
        max_v[pl.ds(16 * k, 16)] = jnp.full((16,), run_max)
    pltpu.sync_copy(max_v, maxes.at[pl.ds(wid * 128, 128)])


@functools.partial(
    pl.kernel,
    out_type=[
        jax.ShapeDtypeStruct((EPAD // EB, EB), jnp.float32),
        jax.ShapeDtypeStruct((NW * 128,), jnp.float32),
    ],
    mesh=_MESH,
    scratch_types=[
        pltpu.VMEM((RPT_A, EB), jnp.int32),
        pltpu.VMEM((RPT_A, EB), jnp.int32),
        pltpu.VMEM((EB, D), jnp.float32),
        pltpu.VMEM((EB, D), jnp.float32),
        pltpu.VMEM((EB, D), jnp.float32),
        pltpu.VMEM((EB, D), jnp.float32),
        pltpu.VMEM((D,), jnp.float32),
        pltpu.VMEM((RPT_A, EB), jnp.float32),
        pltpu.VMEM((128,), jnp.float32),
        pltpu.SemaphoreType.DMA,
        pltpu.SemaphoreType.DMA,
    ],
)
def _logits_kernel(*refs):
    _logits_body(*refs)


def _agg_body(xlcat, logits2d, maxes, src2d, dst2d,
              out,
              accum,
              sidx_v, didx_v, log_v, l_v, ob_v, maxv, nv, sem):
    c = lax.axis_index("c")
    s = lax.axis_index("s")
    coff = c * N

    m = jnp.full((16,), jnp.float32(-1e30))
    for blk in range(NW * 128 // 1024):
        pltpu.sync_copy(maxes.at[pl.ds(blk * 1024, 1024)], maxv)
        for i in range(1024 // 128):
            m = jnp.maximum(m, maxv[pl.ds(128 * i, 16)])
    gmax = _lane_max(m)

    def zrow(r, _):
        for k in range(DW // 16):
            nv[r, pl.ds(16 * k, 16)] = jnp.zeros((16,), jnp.float32)
        return 0
    lax.fori_loop(0, NCHUNK, zrow, 0)
    for t in range(NPT // NCHUNK):
        pltpu.sync_copy(nv, accum.at[pl.ds(s * NPT + t * NCHUNK, NCHUNK)])

    onehot0 = jnp.where(lax.iota(jnp.int32, 16) == 0,
                        jnp.float32(1.0), jnp.float32(0.0))

    plsc.subcore_barrier()

    rowbase = s * RPT_B

    def group_step(g, _):
        gb = rowbase + g * GB
        pltpu.sync_copy(src2d.at[pl.ds(gb, GB)], sidx_v)
        pltpu.sync_copy(dst2d.at[pl.ds(gb, GB)], didx_v)
        pltpu.sync_copy(logits2d.at[pl.ds(gb, GB)], log_v)
        for r in range(GB):
            for k in range(EB // 16):
                sidx_v[r, pl.ds(16 * k, 16)] = sidx_v[r, pl.ds(16 * k, 16)] + coff

        def batch_step(j, _):
            pltpu.async_copy(xlcat.at[sidx_v.at[j]], l_v, sem).wait()

            wvs = [jnp.exp(log_v[j, pl.ds(16 * q, 16)] - gmax)
                   for q in range(EB // 16)]
            for e in range(EB):
                a = wvs[e >> 4][e & 15]
                for k in range(HD // 16):
                    ob_v[e, pl.ds(16 * k, 16)] = l_v[e, pl.ds(16 * k, 16)] * a
                ob_v[e, pl.ds(HD, 16)] = a * onehot0

            pltpu.sync_copy(ob_v, accum.at[didx_v.at[j]], add=True)
            return 0

        lax.fori_loop(0, GB, batch_step, 0)
        return 0

    lax.fori_loop(0, NGRP, group_step, 0)

    plsc.subcore_barrier()

    for t in range(NPT // NCHUNK):
        base = s * NPT + t * NCHUNK
        pltpu.sync_copy(accum.at[pl.ds(base, NCHUNK)], nv)
        pltpu.sync_copy(nv, out.at[pl.ds(c * NPAD + base, NCHUNK)])


@functools.partial(
    pl.kernel,
    out_type=jax.ShapeDtypeStruct((2 * NPAD, DW), jnp.float32),
    mesh=_MESH,
    compiler_params=_SC_PARAMS,
    scratch_types=[
        pltpu.VMEM_SHARED((NPAD, DW), jnp.float32),
        pltpu.VMEM((GB, EB), jnp.int32),
        pltpu.VMEM((GB, EB), jnp.int32),
        pltpu.VMEM((GB, EB), jnp.float32),
        pltpu.VMEM((EB, HD), jnp.float32),
        pltpu.VMEM((EB, DW), jnp.float32),
        pltpu.VMEM((1024,), jnp.float32),
        pltpu.VMEM((NCHUNK, DW), jnp.float32),
        pltpu.SemaphoreType.DMA,
    ],
)
def _agg_kernel(*refs):
    _agg_body(*refs)


def _finish_body(n0_ref, n1_ref, b_ref, out_ref):
    n0 = n0_ref[...]
    n1 = n1_ref[...]
    den = n0[:, HD:HD + 1] + jnp.float32(1e-16)
    out = jnp.concatenate([n0[:, :HD], n1[:, :HD]], axis=1) / den + b_ref[...]
    out_ref[...] = 0.5 * out * (1.0 + lax.erf(out / jnp.sqrt(2.0).astype(jnp.float32)))


def _finish(num0, num1, bias):
    return pl.pallas_call(
        _finish_body,
        grid=(N // ROW_BLK,),
        in_specs=[
            pl.BlockSpec((ROW_BLK, DW), lambda i: (i, 0)),
            pl.BlockSpec((ROW_BLK, DW), lambda i: (i, 0)),
            pl.BlockSpec((D,), lambda i: (0,)),
        ],
        out_specs=pl.BlockSpec((ROW_BLK, D), lambda i: (i, 0)),
        out_shape=jax.ShapeDtypeStruct((N, D), jnp.float32),
    )(num0, num1, bias)


def kernel(x, edge_index, ln_w, ln_b, W_l, W_r, att, bias):
    xl, xr = _ln_proj(x, ln_w, ln_b, W_l, W_r)

    loops = jnp.arange(N, dtype=edge_index.dtype)
    pad = jnp.zeros((EPAD - ETOT,), dtype=edge_index.dtype)
    src2d = jnp.concatenate([edge_index[0], loops, pad]).reshape(EPAD // EB, EB)
    dst2d = jnp.concatenate([edge_index[1], loops, pad]).reshape(EPAD // EB, EB)

    logits2d, maxes = _logits_kernel(xl, xr, att.reshape(D), src2d, dst2d)
    xlcat = jnp.concatenate([xl[:, :HD], xl[:, HD:]], axis=0)
    outc = _agg_kernel(xlcat, logits2d, maxes, src2d, dst2d)
    return _finish(outc[:N], outc[NPAD:NPAD + N], bias)


# speedup vs baseline: 2.9895x; 1.0765x over previous
import functools

import jax
import jax.numpy as jnp
from jax import lax
from jax.experimental import pallas as pl
from jax.experimental.pallas import tpu as pltpu
from jax.experimental.pallas import tpu_sc as plsc

N = 10000
E = 160000
ETOT = E + N
D = 256
HD = 128
ROW_BLK = 400

NC = 2
NS = 16
NW = NC * NS
EB = 64
EPAD = 180224
RPT_A = EPAD // (NW * EB)
RPT_B = EPAD // (NS * EB)
NPAD = 10240
NPT = NPAD // NS
NCHUNK = 32
GB = 8
NGRP = RPT_B // GB
DW = HD + 16
_SC_PARAMS = None

_MESH = plsc.VectorSubcoreMesh(
    core_axis_name="c", subcore_axis_name="s", num_cores=NC, num_subcores=NS)
_SC_PARAMS = pltpu.CompilerParams(use_tc_tiling_on_sc=False)


def _ln_proj_body(x_ref, lnw_ref, lnb_ref, wl_ref, wr_ref, xl_ref, xr_ref):
    xb = x_ref[...]
    mu = jnp.mean(xb, axis=-1, keepdims=True)
    var = jnp.mean((xb - mu) ** 2, axis=-1, keepdims=True)
    xn = (xb - mu) / jnp.sqrt(var + 1e-5) * lnw_ref[...] + lnb_ref[...]
    xl_ref[...] = jnp.dot(xn, wl_ref[...], preferred_element_type=jnp.float32)
    xr_ref[...] = jnp.dot(xn, wr_ref[...], preferred_element_type=jnp.float32)


def _ln_proj(x, ln_w, ln_b, W_l, W_r):
    return pl.pallas_call(
        _ln_proj_body,
        grid=(N // ROW_BLK,),
        in_specs=[
            pl.BlockSpec((ROW_BLK, D), lambda i: (i, 0)),
            pl.BlockSpec((D,), lambda i: (0,)),
            pl.BlockSpec((D,), lambda i: (0,)),
            pl.BlockSpec((D, D), lambda i: (0, 0)),
            pl.BlockSpec((D, D), lambda i: (0, 0)),
        ],
        out_specs=[
            pl.BlockSpec((ROW_BLK, D), lambda i: (i, 0)),
            pl.BlockSpec((ROW_BLK, D), lambda i: (i, 0)),
        ],
        out_shape=[
            jax.ShapeDtypeStruct((N, D), jnp.float32),
            jax.ShapeDtypeStruct((N, D), jnp.float32),
        ],
    )(x, ln_w, ln_b, W_l, W_r)


def _lane_reduce(v, op):
    for sh in (8, 4, 2, 1):
        idx = jnp.arange(16, dtype=jnp.int32) ^ sh
        v = op(v, v.at[idx].get(mode="promise_in_bounds"))
    return v


def _lane_sum(v):
    return _lane_reduce(v, jnp.add)[0]


def _lane_max(v):
    return _lane_reduce(v, jnp.maximum)[0]

def _logits_body(xl, xr, att_hbm, src2d, dst2d,
                 logits2d, maxes,
                 sidx_v, didx_v, l_a, r_a, l_b, r_b,
                 att_v, log_v, max_v, sema, semb):
    c = lax.axis_index("c")
    s = lax.axis_index("s")
    wid = s * NC + c
    rowbase = wid * RPT_A

    pltpu.sync_copy(src2d.at[pl.ds(rowbase, RPT_A)], sidx_v)
    pltpu.sync_copy(dst2d.at[pl.ds(rowbase, RPT_A)], didx_v)
    pltpu.sync_copy(att_hbm, att_v)
    att_regs = [att_v[pl.ds(16 * k, 16)] for k in range(16)]

    lanes = lax.iota(jnp.int32, 16)
    bufs = ((l_a, r_a, sema), (l_b, r_b, semb))

    def fire(j, b):
        lv, rv, sem = bufs[b]
        pltpu.async_copy(xl.at[sidx_v.at[j]], lv, sem)
        pltpu.async_copy(xr.at[didx_v.at[j]], rv, sem)

    def drain(b):
        lv, rv, sem = bufs[b]
        pltpu.make_async_copy(xl.at[sidx_v.at[0]], lv, sem).wait()
        pltpu.make_async_copy(xr.at[didx_v.at[0]], rv, sem).wait()

    def compute(j, b, run_max):
        lbuf, rbuf, _ = bufs[b]
        for g in range(EB // 16):
            def edge_step(e2, lvec):
                e = g * 16 + e2
                acc = jnp.zeros((16,), jnp.float32)
                for k in range(16):
                    sv = lbuf[e, pl.ds(16 * k, 16)] + rbuf[e, pl.ds(16 * k, 16)]
                    sv = jnp.maximum(sv, 0.2 * sv)
                    acc = acc + sv * att_regs[k]
                val = _lane_sum(acc)
                gid = (rowbase + j) * EB + e
                val = jnp.where(gid < ETOT, val, jnp.float32(-1e30))
                return jnp.where(lanes == e2, val, lvec)

            lvec = lax.fori_loop(0, 16, edge_step, jnp.zeros((16,), jnp.float32))
            log_v[j, pl.ds(16 * g, 16)] = lvec
            run_max = jnp.maximum(run_max, _lane_max(lvec))
        return run_max

    fire(jnp.int32(0), 0)

    def outer(i, run_max):
        j0 = 2 * i
        fire(j0 + 1, 1)
        drain(0)
        run_max = compute(j0, 0, run_max)
        fire(jnp.minimum(j0 + 2, RPT_A - 1), 0)
        drain(1)
        run_max = compute(j0 + 1, 1, run_max)
        return run_max

    run_max = lax.fori_loop(0, RPT_A // 2, outer, jnp.float32(-1e30))
    drain(0)

    pltpu.sync_copy(log_v, logits2d.at[pl.ds(rowbase, RPT_A)])
    for k in range(8):
        max_v[pl.ds(16 * k, 16)] = jnp.full((16,), run_max)
    pltpu.sync_copy(max_v, maxes.at[pl.ds(wid * 128, 128)])


@functools.partial(
    pl.kernel,
    out_type=[
        jax.ShapeDtypeStruct((EPAD // EB, EB), jnp.float32),
        jax.ShapeDtypeStruct((NW * 128,), jnp.float32),
    ],
    mesh=_MESH,
    scratch_types=[
        pltpu.VMEM((RPT_A, EB), jnp.int32),
        pltpu.VMEM((RPT_A, EB), jnp.int32),
        pltpu.VMEM((EB, D), jnp.float32),
        pltpu.VMEM((EB, D), jnp.float32),
        pltpu.VMEM((EB, D), jnp.float32),
        pltpu.VMEM((EB, D), jnp.float32),
        pltpu.VMEM((D,), jnp.float32),
        pltpu.VMEM((RPT_A, EB), jnp.float32),
        pltpu.VMEM((128,), jnp.float32),
        pltpu.SemaphoreType.DMA,
        pltpu.SemaphoreType.DMA,
    ],
)
def _logits_kernel(*refs):
    _logits_body(*refs)


def _agg_body(xlcat, logits2d, maxes, src2d, dst2d, zrows,
              out,
              accum,
              sidx_v, didx_v, log_v, l_a, l_b, ob_v, maxv, sema, semb):
    c = lax.axis_index("c")
    s = lax.axis_index("s")
    coff = c * N

    m = jnp.full((16,), jnp.float32(-1e30))
    for blk in range(NW * 128 // 1024):
        pltpu.sync_copy(maxes.at[pl.ds(blk * 1024, 1024)], maxv)
        for i in range(1024 // 128):
            m = jnp.maximum(m, maxv[pl.ds(128 * i, 16)])
    gmax = _lane_max(m)

    pltpu.sync_copy(zrows, accum.at[pl.ds(s * NPT, NPT)])

    onehot0 = jnp.where(lax.iota(jnp.int32, 16) == 0,
                        jnp.float32(1.0), jnp.float32(0.0))

    plsc.subcore_barrier()

    rowbase = s * RPT_B
    gbufs = ((l_a, sema), (l_b, semb))

    def fire(j, b):
        lv, sem = gbufs[b]
        pltpu.async_copy(xlcat.at[sidx_v.at[j]], lv, sem)

    def drain(b):
        lv, sem = gbufs[b]
        pltpu.make_async_copy(xlcat.at[sidx_v.at[0]], lv, sem).wait()

    def compute_scatter(j, b):
        lv, _ = gbufs[b]
        wvs = [jnp.exp(log_v[j, pl.ds(16 * q, 16)] - gmax)
               for q in range(EB // 16)]
        for e in range(EB):
            a = wvs[e >> 4][e & 15]
            for k in range(HD // 16):
                ob_v[e, pl.ds(16 * k, 16)] = lv[e, pl.ds(16 * k, 16)] * a
            ob_v[e, pl.ds(HD, 16)] = a * onehot0
        pltpu.sync_copy(ob_v, accum.at[didx_v.at[j]], add=True)

    def group_step(g, _):
        gb = rowbase + g * GB
        pltpu.sync_copy(src2d.at[pl.ds(gb, GB)], sidx_v)
        pltpu.sync_copy(dst2d.at[pl.ds(gb, GB)], didx_v)
        pltpu.sync_copy(logits2d.at[pl.ds(gb, GB)], log_v)
        for r in range(GB):
            for k in range(EB // 16):
                sidx_v[r, pl.ds(16 * k, 16)] = sidx_v[r, pl.ds(16 * k, 16)] + coff

        fire(jnp.int32(0), 0)

        def pair_step(i, _):
            j0 = 2 * i
            fire(j0 + 1, 1)
            drain(0)
            compute_scatter(j0, 0)
            fire(jnp.minimum(j0 + 2, GB - 1), 0)
            drain(1)
            compute_scatter(j0 + 1, 1)
            return 0

        lax.fori_loop(0, GB // 2, pair_step, 0)
        drain(0)
        return 0

    lax.fori_loop(0, NGRP, group_step, 0)

    plsc.subcore_barrier()

    pltpu.sync_copy(accum.at[pl.ds(s * NPT, NPT)],
                    out.at[pl.ds(c * NPAD + s * NPT, NPT)])


@functools.partial(
    pl.kernel,
    out_type=jax.ShapeDtypeStruct((2 * NPAD, DW), jnp.float32),
    mesh=_MESH,
    compiler_params=_SC_PARAMS,
    scratch_types=[
        pltpu.VMEM_SHARED((NPAD, DW), jnp.float32),
        pltpu.VMEM((GB, EB), jnp.int32),
        pltpu.VMEM((GB, EB), jnp.int32),
        pltpu.VMEM((GB, EB), jnp.float32),
        pltpu.VMEM((EB, HD), jnp.float32),
        pltpu.VMEM((EB, HD), jnp.float32),
        pltpu.VMEM((EB, DW), jnp.float32),
        pltpu.VMEM((1024,), jnp.float32),
        pltpu.SemaphoreType.DMA,
        pltpu.SemaphoreType.DMA,
    ],
)
def _agg_kernel(*refs):
    _agg_body(*refs)


def _finish_body(n0_ref, n1_ref, b_ref, out_ref):
    n0 = n0_ref[...]
    n1 = n1_ref[...]
    den = n0[:, HD:HD + 1] + jnp.float32(1e-16)
    out = jnp.concatenate([n0[:, :HD], n1[:, :HD]], axis=1) / den + b_ref[...]
    out_ref[...] = 0.5 * out * (1.0 + lax.erf(out / jnp.sqrt(2.0).astype(jnp.float32)))


def _finish(num0, num1, bias):
    return pl.pallas_call(
        _finish_body,
        grid=(N // ROW_BLK,),
        in_specs=[
            pl.BlockSpec((ROW_BLK, DW), lambda i: (i, 0)),
            pl.BlockSpec((ROW_BLK, DW), lambda i: (i, 0)),
            pl.BlockSpec((D,), lambda i: (0,)),
        ],
        out_specs=pl.BlockSpec((ROW_BLK, D), lambda i: (i, 0)),
        out_shape=jax.ShapeDtypeStruct((N, D), jnp.float32),
    )(num0, num1, bias)


def kernel(x, edge_index, ln_w, ln_b, W_l, W_r, att, bias):
    xl, xr = _ln_proj(x, ln_w, ln_b, W_l, W_r)

    loops = jnp.arange(N, dtype=edge_index.dtype)
    pad = jnp.zeros((EPAD - ETOT,), dtype=edge_index.dtype)
    src2d = jnp.concatenate([edge_index[0], loops, pad]).reshape(EPAD // EB, EB)
    dst2d = jnp.concatenate([edge_index[1], loops, pad]).reshape(EPAD // EB, EB)

    logits2d, maxes = _logits_kernel(xl, xr, att.reshape(D), src2d, dst2d)
    xlcat = jnp.concatenate([xl[:, :HD], xl[:, HD:]], axis=0)
    zrows = jnp.zeros((NPT, DW), jnp.float32)
    outc = _agg_kernel(xlcat, logits2d, maxes, src2d, dst2d, zrows)
    return _finish(outc[:N], outc[NPAD:NPAD + N], bias)


# speedup vs baseline: 3.0052x; 1.0052x over previous
import functools

import jax
import jax.numpy as jnp
from jax import lax
from jax.experimental import pallas as pl
from jax.experimental.pallas import tpu as pltpu
from jax.experimental.pallas import tpu_sc as plsc

N = 10000
E = 160000
ETOT = E + N
D = 256
HD = 128
ROW_BLK = 400

NC = 2
NS = 16
NW = NC * NS
EB = 64
EPAD = 180224
RPT_A = EPAD // (NW * EB)
RPT_B = EPAD // (NS * EB)
NPAD = 10240
NPT = NPAD // NS
NCHUNK = 32
GB = 8
NGRP = RPT_B // GB
DW = HD + 16
_SC_PARAMS = None

_MESH = plsc.VectorSubcoreMesh(
    core_axis_name="c", subcore_axis_name="s", num_cores=NC, num_subcores=NS)
_SC_PARAMS = pltpu.CompilerParams(use_tc_tiling_on_sc=False)


def _ln_proj_body(x_ref, lnw_ref, lnb_ref, wl_ref, wr_ref, xl_ref, xr_ref):
    xb = x_ref[...]
    mu = jnp.mean(xb, axis=-1, keepdims=True)
    var = jnp.mean((xb - mu) ** 2, axis=-1, keepdims=True)
    xn = (xb - mu) / jnp.sqrt(var + 1e-5) * lnw_ref[...] + lnb_ref[...]
    xl_ref[...] = jnp.dot(xn, wl_ref[...], preferred_element_type=jnp.float32)
    xr_ref[...] = jnp.dot(xn, wr_ref[...], preferred_element_type=jnp.float32)


def _ln_proj(x, ln_w, ln_b, W_l, W_r):
    return pl.pallas_call(
        _ln_proj_body,
        grid=(N // ROW_BLK,),
        in_specs=[
            pl.BlockSpec((ROW_BLK, D), lambda i: (i, 0)),
            pl.BlockSpec((D,), lambda i: (0,)),
            pl.BlockSpec((D,), lambda i: (0,)),
            pl.BlockSpec((D, D), lambda i: (0, 0)),
            pl.BlockSpec((D, D), lambda i: (0, 0)),
        ],
        out_specs=[
            pl.BlockSpec((ROW_BLK, D), lambda i: (i, 0)),
            pl.BlockSpec((ROW_BLK, D), lambda i: (i, 0)),
        ],
        out_shape=[
            jax.ShapeDtypeStruct((N, D), jnp.float32),
            jax.ShapeDtypeStruct((N, D), jnp.float32),
        ],
    )(x, ln_w, ln_b, W_l, W_r)


def _lane_reduce(v, op):
    for sh in (8, 4, 2, 1):
        idx = jnp.arange(16, dtype=jnp.int32) ^ sh
        v = op(v, v.at[idx].get(mode="promise_in_bounds"))
    return v


def _lane_sum(v):
    return _lane_reduce(v, jnp.add)[0]


def _lane_max(v):
    return _lane_reduce(v, jnp.maximum)[0]

def _logits_body(xl, xr, att_hbm, src2d, dst2d,
                 logits2d, maxes,
                 sidx_v, didx_v, l_a, r_a, l_b, r_b,
                 att_v, log_v, max_v, sema, semb):
    c = lax.axis_index("c")
    s = lax.axis_index("s")
    wid = s * NC + c
    rowbase = wid * RPT_A

    pltpu.sync_copy(src2d.at[pl.ds(rowbase, RPT_A)], sidx_v)
    pltpu.sync_copy(dst2d.at[pl.ds(rowbase, RPT_A)], didx_v)
    pltpu.sync_copy(att_hbm, att_v)
    att_regs = [att_v[pl.ds(16 * k, 16)] for k in range(16)]

    lanes = lax.iota(jnp.int32, 16)
    bufs = ((l_a, r_a, sema), (l_b, r_b, semb))

    def fire(j, b):
        lv, rv, sem = bufs[b]
        pltpu.async_copy(xl.at[sidx_v.at[j]], lv, sem)
        pltpu.async_copy(xr.at[didx_v.at[j]], rv, sem)

    def drain(b):
        lv, rv, sem = bufs[b]
        pltpu.make_async_copy(xl.at[sidx_v.at[0]], lv, sem).wait()
        pltpu.make_async_copy(xr.at[didx_v.at[0]], rv, sem).wait()

    def compute(j, b, run_max):
        lbuf, rbuf, _ = bufs[b]
        for g in range(EB // 16):
            def edge_step(e2, lvec):
                e = g * 16 + e2
                acc0 = jnp.zeros((16,), jnp.float32)
                acc1 = jnp.zeros((16,), jnp.float32)
                for k in range(0, 16, 2):
                    sv0 = lbuf[e, pl.ds(16 * k, 16)] + rbuf[e, pl.ds(16 * k, 16)]
                    sv1 = lbuf[e, pl.ds(16 * (k + 1), 16)] + rbuf[e, pl.ds(16 * (k + 1), 16)]
                    sv0 = jnp.maximum(sv0, 0.2 * sv0)
                    sv1 = jnp.maximum(sv1, 0.2 * sv1)
                    acc0 = acc0 + sv0 * att_regs[k]
                    acc1 = acc1 + sv1 * att_regs[k + 1]
                val = _lane_sum(acc0 + acc1)
                gid = (rowbase + j) * EB + e
                val = jnp.where(gid < ETOT, val, jnp.float32(-1e30))
                return jnp.where(lanes == e2, val, lvec)

            lvec = lax.fori_loop(0, 16, edge_step, jnp.zeros((16,), jnp.float32))
            log_v[j, pl.ds(16 * g, 16)] = lvec
            run_max = jnp.maximum(run_max, _lane_max(lvec))
        return run_max

    fire(jnp.int32(0), 0)

    def outer(i, run_max):
        j0 = 2 * i
        fire(j0 + 1, 1)
        drain(0)
        run_max = compute(j0, 0, run_max)
        fire(jnp.minimum(j0 + 2, RPT_A - 1), 0)
        drain(1)
        run_max = compute(j0 + 1, 1, run_max)
        return run_max

    run_max = lax.fori_loop(0, RPT_A // 2, outer, jnp.float32(-1e30))
    drain(0)

    pltpu.sync_copy(log_v, logits2d.at[pl.ds(rowbase, RPT_A)])
    for k in range(8):
        max_v[pl.ds(16 * k, 16)] = jnp.full((16,), run_max)
    pltpu.sync_copy(max_v, maxes.at[pl.ds(wid * 128, 128)])


@functools.partial(
    pl.kernel,
    out_type=[
        jax.ShapeDtypeStruct((EPAD // EB, EB), jnp.float32),
        jax.ShapeDtypeStruct((NW * 128,), jnp.float32),
    ],
    mesh=_MESH,
    scratch_types=[
        pltpu.VMEM((RPT_A, EB), jnp.int32),
        pltpu.VMEM((RPT_A, EB), jnp.int32),
        pltpu.VMEM((EB, D), jnp.float32),
        pltpu.VMEM((EB, D), jnp.float32),
        pltpu.VMEM((EB, D), jnp.float32),
        pltpu.VMEM((EB, D), jnp.float32),
        pltpu.VMEM((D,), jnp.float32),
        pltpu.VMEM((RPT_A, EB), jnp.float32),
        pltpu.VMEM((128,), jnp.float32),
        pltpu.SemaphoreType.DMA,
        pltpu.SemaphoreType.DMA,
    ],
)
def _logits_kernel(*refs):
    _logits_body(*refs)


def _agg_body(xlcat, logits2d, maxes, src2d, dst32, zrows,
              out,
              accum,
              sidx_v, didx_v, log_v, l_a, l_b, ob_a, ob_b,
              maxv, sema, semb, sem_s):
    c = lax.axis_index("c")
    s = lax.axis_index("s")
    coff = c * N

    m = jnp.full((16,), jnp.float32(-1e30))
    for blk in range(NW * 128 // 1024):
        pltpu.sync_copy(maxes.at[pl.ds(blk * 1024, 1024)], maxv)
        for i in range(1024 // 128):
            m = jnp.maximum(m, maxv[pl.ds(128 * i, 16)])
    gmax = _lane_max(m)

    pltpu.sync_copy(zrows, accum.at[pl.ds(s * NPT, NPT)])

    onehot0 = jnp.where(lax.iota(jnp.int32, 16) == 0,
                        jnp.float32(1.0), jnp.float32(0.0))

    plsc.subcore_barrier()

    rowbase = s * RPT_B
    gbufs = ((l_a, sema), (l_b, semb))

    def fire(j, b):
        lv, sem = gbufs[b]
        pltpu.async_copy(xlcat.at[sidx_v.at[j]], lv, sem)

    def drain(b):
        lv, sem = gbufs[b]
        pltpu.make_async_copy(xlcat.at[sidx_v.at[0]], lv, sem).wait()

    def compute_half(j, b, h, obuf):
        lv, _ = gbufs[b]
        wvs = [jnp.exp(log_v[j, pl.ds(16 * q, 16)] - gmax)
               for q in range(2 * h, 2 * h + 2)]
        for e2 in range(EB // 2):
            e = 32 * h + e2
            a = wvs[e2 >> 4][e2 & 15]
            for k in range(HD // 16):
                obuf[e2, pl.ds(16 * k, 16)] = lv[e, pl.ds(16 * k, 16)] * a
            obuf[e2, pl.ds(HD, 16)] = a * onehot0
        pltpu.async_copy(obuf, accum.at[didx_v.at[2 * j + h]], sem_s, add=True)

    def compute_scatter(j, b):
        compute_half(j, b, 0, ob_a)
        compute_half(j, b, 1, ob_b)
        pltpu.make_async_copy(ob_a, accum.at[didx_v.at[0]], sem_s).wait()
        pltpu.make_async_copy(ob_b, accum.at[didx_v.at[0]], sem_s).wait()

    def group_step(g, _):
        gb = rowbase + g * GB
        pltpu.sync_copy(src2d.at[pl.ds(gb, GB)], sidx_v)
        pltpu.sync_copy(dst32.at[pl.ds(2 * gb, 2 * GB)], didx_v)
        pltpu.sync_copy(logits2d.at[pl.ds(gb, GB)], log_v)
        for r in range(GB):
            for k in range(EB // 16):
                sidx_v[r, pl.ds(16 * k, 16)] = sidx_v[r, pl.ds(16 * k, 16)] + coff

        fire(jnp.int32(0), 0)

        def pair_step(i, _):
            j0 = 2 * i
            fire(j0 + 1, 1)
            drain(0)
            compute_scatter(j0, 0)
            fire(jnp.minimum(j0 + 2, GB - 1), 0)
            drain(1)
            compute_scatter(j0 + 1, 1)
            return 0

        lax.fori_loop(0, GB // 2, pair_step, 0)
        drain(0)
        return 0

    lax.fori_loop(0, NGRP, group_step, 0)

    plsc.subcore_barrier()

    pltpu.sync_copy(accum.at[pl.ds(s * NPT, NPT)],
                    out.at[pl.ds(c * NPAD + s * NPT, NPT)])


@functools.partial(
    pl.kernel,
    out_type=jax.ShapeDtypeStruct((2 * NPAD, DW), jnp.float32),
    mesh=_MESH,
    compiler_params=_SC_PARAMS,
    scratch_types=[
        pltpu.VMEM_SHARED((NPAD, DW), jnp.float32),
        pltpu.VMEM((GB, EB), jnp.int32),
        pltpu.VMEM((2 * GB, EB // 2), jnp.int32),
        pltpu.VMEM((GB, EB), jnp.float32),
        pltpu.VMEM((EB, HD), jnp.float32),
        pltpu.VMEM((EB, HD), jnp.float32),
        pltpu.VMEM((EB // 2, DW), jnp.float32),
        pltpu.VMEM((EB // 2, DW), jnp.float32),
        pltpu.VMEM((1024,), jnp.float32),
        pltpu.SemaphoreType.DMA,
        pltpu.SemaphoreType.DMA,
        pltpu.SemaphoreType.DMA,
    ],
)
def _agg_kernel(*refs):
    _agg_body(*refs)


def _finish_body(n0_ref, n1_ref, b_ref, out_ref):
    n0 = n0_ref[...]
    n1 = n1_ref[...]
    den = n0[:, HD:HD + 1] + jnp.float32(1e-16)
    out = jnp.concatenate([n0[:, :HD], n1[:, :HD]], axis=1) / den + b_ref[...]
    out_ref[...] = 0.5 * out * (1.0 + lax.erf(out / jnp.sqrt(2.0).astype(jnp.float32)))


def _finish(num0, num1, bias):
    return pl.pallas_call(
        _finish_body,
        grid=(N // ROW_BLK,),
        in_specs=[
            pl.BlockSpec((ROW_BLK, DW), lambda i: (i, 0)),
            pl.BlockSpec((ROW_BLK, DW), lambda i: (i, 0)),
            pl.BlockSpec((D,), lambda i: (0,)),
        ],
        out_specs=pl.BlockSpec((ROW_BLK, D), lambda i: (i, 0)),
        out_shape=jax.ShapeDtypeStruct((N, D), jnp.float32),
    )(num0, num1, bias)


def kernel(x, edge_index, ln_w, ln_b, W_l, W_r, att, bias):
    xl, xr = _ln_proj(x, ln_w, ln_b, W_l, W_r)

    loops = jnp.arange(N, dtype=edge_index.dtype)
    pad = jnp.zeros((EPAD - ETOT,), dtype=edge_index.dtype)
    src2d = jnp.concatenate([edge_index[0], loops, pad]).reshape(EPAD // EB, EB)
    dst2d = jnp.concatenate([edge_index[1], loops, pad]).reshape(EPAD // EB, EB)

    logits2d, maxes = _logits_kernel(xl, xr, att.reshape(D), src2d, dst2d)
    xlcat = jnp.concatenate([xl[:, :HD], xl[:, HD:]], axis=0)
    zrows = jnp.zeros((NPT, DW), jnp.float32)
    dst32 = dst2d.reshape(2 * (EPAD // EB), EB // 2)
    outc = _agg_kernel(xlcat, logits2d, maxes, src2d, dst32, zrows)
    return _finish(outc[:N], outc[NPAD:NPAD + N], bias)
